# baseline (device time: 26127 ns/iter reference)
import jax
import jax.numpy as jnp
from jax import lax
from jax.experimental import pallas as pl
from jax.experimental.pallas import tpu as pltpu


def kernel(dy, W):
    M, K = dy.shape
    D = W.shape[0]
    QR = M // 4

    def body(dy_ref, w_ref, out_ref, recv_x, send_sems, recv_sems):
        x = lax.axis_index("x")
        y = lax.axis_index("y")
        z = lax.axis_index("z")
        m = 2 * z + y
        row0 = m * QR

        barrier = pltpu.get_barrier_semaphore()
        for dev in [(1 - x, y, z), (x, 1 - y, z), (x, y, 1 - z)]:
            pl.semaphore_signal(
                barrier, inc=1, device_id=dev,
                device_id_type=pl.DeviceIdType.MESH,
            )
        pl.semaphore_wait(barrier, 3)

        dyq = dy_ref[pl.ds(row0, QR), :]
        pq = lax.dot_general(
            dyq, w_ref[:, :],
            (((1,), (1,)), ((), ())),
            preferred_element_type=jnp.float32,
        )
        out_ref[pl.ds(row0, QR), :] = pq

        rdma_x = pltpu.make_async_remote_copy(
            src_ref=out_ref.at[pl.ds(row0, QR)],
            dst_ref=recv_x,
            send_sem=send_sems.at[0],
            recv_sem=recv_sems.at[0],
            device_id=(1 - x, y, z),
            device_id_type=pl.DeviceIdType.MESH,
        )
        rdma_x.start()
        rdma_x.wait()
        out_ref[pl.ds(row0, QR), :] = pq + recv_x[:, :]

        rdma_y = pltpu.make_async_remote_copy(
            src_ref=out_ref.at[pl.ds(row0, QR)],
            dst_ref=out_ref.at[pl.ds(row0, QR)],
            send_sem=send_sems.at[1],
            recv_sem=recv_sems.at[1],
            device_id=(x, 1 - y, z),
            device_id_type=pl.DeviceIdType.MESH,
        )
        rdma_y.start()
        rdma_y.wait()

        half0 = z * (2 * QR)
        rdma_z = pltpu.make_async_remote_copy(
            src_ref=out_ref.at[pl.ds(half0, 2 * QR)],
            dst_ref=out_ref.at[pl.ds(half0, 2 * QR)],
            send_sem=send_sems.at[2],
            recv_sem=recv_sems.at[2],
            device_id=(x, y, 1 - z),
            device_id_type=pl.DeviceIdType.MESH,
        )
        rdma_z.start()
        rdma_z.wait()

    return pl.pallas_call(
        body,
        out_shape=jax.ShapeDtypeStruct((M, D), jnp.float32),
        in_specs=[
            pl.BlockSpec(memory_space=pltpu.VMEM),
            pl.BlockSpec(memory_space=pltpu.VMEM),
        ],
        out_specs=pl.BlockSpec(memory_space=pltpu.VMEM),
        scratch_shapes=[
            pltpu.VMEM((QR, D), jnp.float32),
            pltpu.SemaphoreType.DMA((3,)),
            pltpu.SemaphoreType.DMA((3,)),
        ],
        compiler_params=pltpu.CompilerParams(collective_id=0),
    )(dy, W)


# device time: 17957 ns/iter; 1.4550x vs baseline; 1.4550x over previous
import jax
import jax.numpy as jnp
from jax import lax
from jax.experimental import pallas as pl
from jax.experimental.pallas import tpu as pltpu


def kernel(dy, W):
    M, K = dy.shape
    D = W.shape[0]
    QR = M // 4
    CR = QR // 2

    def body(dy_ref, w_ref, out_ref, rx_own, rx_diag,
             sx, rxs, sy, ry, sz, rz):
        x = lax.axis_index("x")
        y = lax.axis_index("y")
        z = lax.axis_index("z")
        own = 2 * z + y
        diag = 3 - own
        y_own = 2 * z + (1 - y)
        z_own = 2 * (1 - z) + y
        o_row = own * QR
        d_row = diag * QR

        xp = (1 - x, y, z)
        yp = (x, 1 - y, z)
        zp = (x, y, 1 - z)

        barrier = pltpu.get_barrier_semaphore()
        for dev in [xp, yp, zp]:
            pl.semaphore_signal(
                barrier, inc=1, device_id=dev,
                device_id_type=pl.DeviceIdType.MESH,
            )
        pl.semaphore_wait(barrier, 3)

        def quarter_partial(row0):
            dyq = dy_ref[pl.ds(row0, QR), :]
            return lax.dot_general(
                dyq, w_ref[:, :],
                (((1,), (1,)), ((), ())),
                preferred_element_type=jnp.float32,
            )

        pq_own = quarter_partial(o_row)
        out_ref[pl.ds(o_row, QR), :] = pq_own
        x_rdmas = []
        for c in range(2):
            r = pltpu.make_async_remote_copy(
                src_ref=out_ref.at[pl.ds(o_row + c * CR, CR)],
                dst_ref=rx_own.at[pl.ds(c * CR, CR)],
                send_sem=sx.at[c],
                recv_sem=rxs.at[c],
                device_id=xp,
                device_id_type=pl.DeviceIdType.MESH,
            )
            r.start()
            x_rdmas.append(r)

        pq_diag = quarter_partial(d_row)
        out_ref[pl.ds(d_row, QR), :] = pq_diag
        r_diag = pltpu.make_async_remote_copy(
            src_ref=out_ref.at[pl.ds(d_row, QR)],
            dst_ref=rx_diag,
            send_sem=sx.at[2],
            recv_sem=rxs.at[2],
            device_id=xp,
            device_id_type=pl.DeviceIdType.MESH,
        )
        r_diag.start()

        yz_rdmas = []
        for c in range(2):
            x_rdmas[c].wait()
            out_ref[pl.ds(o_row + c * CR, CR), :] = (
                pq_own[c * CR:(c + 1) * CR, :] + rx_own[c * CR:(c + 1) * CR, :]
            )
            for sem_s, sem_r, dev in ((sy, ry, yp), (sz, rz, zp)):
                r = pltpu.make_async_remote_copy(
                    src_ref=out_ref.at[pl.ds(o_row + c * CR, CR)],
                    dst_ref=out_ref.at[pl.ds(o_row + c * CR, CR)],
                    send_sem=sem_s.at[c],
                    recv_sem=sem_r.at[c],
                    device_id=dev,
                    device_id_type=pl.DeviceIdType.MESH,
                )
                r.start()
                yz_rdmas.append(r)

        r_diag.wait()
        out_ref[pl.ds(d_row, QR), :] = pq_diag + rx_diag[:, :]

        for sem_s, sem_r, dev, src_q in (
            (sy, ry, yp, y_own),
            (sz, rz, zp, z_own),
        ):
            for c in range(2):
                r = pltpu.make_async_remote_copy(
                    src_ref=out_ref.at[pl.ds(src_q * QR + c * CR, CR)],
                    dst_ref=out_ref.at[pl.ds(src_q * QR + c * CR, CR)],
                    send_sem=sem_s.at[c],
                    recv_sem=sem_r.at[c],
                    device_id=dev,
                    device_id_type=pl.DeviceIdType.MESH,
                )
                r.wait_recv()

        for r in yz_rdmas:
            r.wait_send()

    return pl.pallas_call(
        body,
        out_shape=jax.ShapeDtypeStruct((M, D), jnp.float32),
        in_specs=[
            pl.BlockSpec(memory_space=pltpu.VMEM),
            pl.BlockSpec(memory_space=pltpu.VMEM),
        ],
        out_specs=pl.BlockSpec(memory_space=pltpu.VMEM),
        scratch_shapes=[
            pltpu.VMEM((QR, D), jnp.float32),
            pltpu.VMEM((QR, D), jnp.float32),
            pltpu.SemaphoreType.DMA((3,)),
            pltpu.SemaphoreType.DMA((3,)),
            pltpu.SemaphoreType.DMA((2,)),
            pltpu.SemaphoreType.DMA((2,)),
            pltpu.SemaphoreType.DMA((2,)),
            pltpu.SemaphoreType.DMA((2,)),
        ],
        compiler_params=pltpu.CompilerParams(collective_id=0),
    )(dy, W)
